# baseline (speedup 1.0000x reference)
import functools

import jax
import jax.numpy as jnp
from jax import lax
from jax.experimental import pallas as pl
from jax.experimental.pallas import tpu as pltpu
from jax.experimental.pallas import tpu_sc as plsc

T, D = 2048, 1024
NE, FF = 8, 512
TAU = 0.5

RBLK = 256
MBLK = 128
NBLK = 2 * T // MBLK + NE
NPAD = NBLK * MBLK
NW = 32
ROWS_W = NPAD // NW
GCH = 32
CTOK = T // NW
CCH = 16


def _router_body(x_ref, gw_ref, ngw_ref, gum_ref, sel_ref, w_ref):
    xh = x_ref[...].astype(jnp.bfloat16)
    cdims = (((1,), (1,)), ((), ()))
    ol = jax.lax.dot_general(xh, gw_ref[...], cdims,
                             preferred_element_type=jnp.float32)
    rl = jax.lax.dot_general(xh, ngw_ref[...], cdims,
                             preferred_element_type=jnp.float32)
    rwts = jax.nn.softmax(ol, axis=-1)
    gl = (rl + gum_ref[...]) * (1.0 / TAU)

    ids = jax.lax.broadcasted_iota(jnp.int32, (RBLK, NE), 1)
    m1 = jnp.max(gl, axis=1, keepdims=True)
    i1 = jnp.min(jnp.where(gl == m1, ids, NE), axis=1, keepdims=True)
    gl2 = jnp.where(ids == i1, -1e30, gl)
    m2 = jnp.max(gl2, axis=1, keepdims=True)
    i2 = jnp.min(jnp.where(gl2 == m2, ids, NE), axis=1, keepdims=True)
    w1 = jnp.sum(jnp.where(ids == i1, rwts, 0.0), axis=1, keepdims=True)
    w2 = jnp.sum(jnp.where(ids == i2, rwts, 0.0), axis=1, keepdims=True)
    sel_ref[...] = jnp.concatenate([i1, i2], axis=1)
    w_ref[...] = jnp.concatenate([w1, w2], axis=1)


def _router(x2d, gwh, ngwh, gum):
    return pl.pallas_call(
        _router_body,
        grid=(T // RBLK,),
        in_specs=[
            pl.BlockSpec((RBLK, D), lambda i: (i, 0)),
            pl.BlockSpec((NE, D), lambda i: (0, 0)),
            pl.BlockSpec((NE, D), lambda i: (0, 0)),
            pl.BlockSpec((RBLK, NE), lambda i: (i, 0)),
        ],
        out_specs=[
            pl.BlockSpec((RBLK, 2), lambda i: (i, 0)),
            pl.BlockSpec((RBLK, 2), lambda i: (i, 0)),
        ],
        out_shape=[
            jax.ShapeDtypeStruct((T, 2), jnp.int32),
            jax.ShapeDtypeStruct((T, 2), jnp.float32),
        ],
    )(x2d, gwh, ngwh, gum)


@functools.cache
def _get_gather_sc():
    mesh = plsc.VectorSubcoreMesh(core_axis_name="c", subcore_axis_name="s")

    @functools.partial(
        pl.kernel,
        mesh=mesh,
        out_type=jax.ShapeDtypeStruct((NPAD, D), jnp.float32),
        scratch_types=[
            pltpu.VMEM((GCH,), jnp.int32),
            pltpu.VMEM((GCH, D), jnp.float32),
            pltpu.SemaphoreType.DMA,
        ],
    )
    def _gather_sc(x_hbm, gidx_hbm, out_hbm, idx_v, rows_v, sem):
        wid = lax.axis_index("s") * 2 + lax.axis_index("c")
        base = wid * ROWS_W
        for c in range(ROWS_W // GCH):
            s = base + c * GCH
            pltpu.sync_copy(gidx_hbm.at[pl.ds(s, GCH)], idx_v)
            pltpu.async_copy(x_hbm.at[idx_v], rows_v, sem).wait()
            pltpu.sync_copy(rows_v, out_hbm.at[pl.ds(s, GCH)])

    return _gather_sc


def _mlp_body(be_ref, xg_ref, gwp_ref, eg_ref, eu_ref, ed_ref, out_ref):
    del be_ref
    xh = xg_ref[...].astype(jnp.bfloat16)
    cdims = (((1,), (1,)), ((), ()))
    g = jax.lax.dot_general(xh, eg_ref[0], cdims,
                            preferred_element_type=jnp.float32)
    u = jax.lax.dot_general(xh, eu_ref[0], cdims,
                            preferred_element_type=jnp.float32)
    h = (g * jax.nn.sigmoid(g) * u).astype(jnp.bfloat16)
    o = jax.lax.dot_general(h, ed_ref[0], cdims,
                            preferred_element_type=jnp.float32)
    out_ref[...] = o * gwp_ref[0]


def _mlp(block_expert, xg, gwp, egh, euh, edh):
    grid_spec = pltpu.PrefetchScalarGridSpec(
        num_scalar_prefetch=1,
        grid=(NBLK,),
        in_specs=[
            pl.BlockSpec((MBLK, D), lambda i, be: (i, 0)),
            pl.BlockSpec((1, MBLK, 1), lambda i, be: (i, 0, 0)),
            pl.BlockSpec((1, FF, D), lambda i, be: (be[i], 0, 0)),
            pl.BlockSpec((1, FF, D), lambda i, be: (be[i], 0, 0)),
            pl.BlockSpec((1, D, FF), lambda i, be: (be[i], 0, 0)),
        ],
        out_specs=pl.BlockSpec((MBLK, D), lambda i, be: (i, 0)),
    )
    return pl.pallas_call(
        _mlp_body,
        grid_spec=grid_spec,
        out_shape=jax.ShapeDtypeStruct((NPAD, D), jnp.float32),
    )(block_expert, xg, gwp, egh, euh, edh)


@functools.cache
def _get_combine_sc():
    mesh = plsc.VectorSubcoreMesh(core_axis_name="c", subcore_axis_name="s")

    @functools.partial(
        pl.kernel,
        mesh=mesh,
        out_type=jax.ShapeDtypeStruct((T, D), jnp.float32),
        scratch_types=[
            pltpu.VMEM((CCH,), jnp.int32),
            pltpu.VMEM((CCH, D), jnp.float32),
            pltpu.VMEM((CCH, D), jnp.float32),
            pltpu.SemaphoreType.DMA,
        ],
    )
    def _combine_sc(contrib_hbm, pos1_hbm, pos2_hbm, out_hbm, idx_v, a_v,
                    b_v, sem):
        wid = lax.axis_index("s") * 2 + lax.axis_index("c")
        base = wid * CTOK
        for c in range(CTOK // CCH):
            s = base + c * CCH
            pltpu.sync_copy(pos1_hbm.at[pl.ds(s, CCH)], idx_v)
            pltpu.async_copy(contrib_hbm.at[idx_v], a_v, sem).wait()
            pltpu.sync_copy(pos2_hbm.at[pl.ds(s, CCH)], idx_v)
            pltpu.async_copy(contrib_hbm.at[idx_v], b_v, sem).wait()

            def _add(i, _):
                j = i // (D // 16)
                k = (i % (D // 16)) * 16
                a_v[j, pl.ds(k, 16)] = (a_v[j, pl.ds(k, 16)]
                                        + b_v[j, pl.ds(k, 16)])
                return 0

            lax.fori_loop(0, CCH * (D // 16), _add, 0)
            pltpu.sync_copy(a_v, out_hbm.at[pl.ds(s, CCH)])

    return _combine_sc


def kernel(hidden_states, gate_w, new_gate_w, expert_gate_w, expert_up_w,
           expert_down_w):
    x2d = hidden_states.reshape(T, D)
    gum = jax.random.gumbel(jax.random.key(42), (T, NE), dtype=jnp.float32)
    gwh = gate_w.astype(jnp.bfloat16)
    ngwh = new_gate_w.astype(jnp.bfloat16)
    egh = expert_gate_w.astype(jnp.bfloat16)
    euh = expert_up_w.astype(jnp.bfloat16)
    edh = expert_down_w.astype(jnp.bfloat16)

    sel, w = _router(x2d, gwh, ngwh, gum)

    ef = sel.reshape(-1)
    onehot = (ef[:, None] == jnp.arange(NE)[None, :]).astype(jnp.int32)
    csum = jnp.cumsum(onehot, axis=0)
    counts = csum[-1]
    rank = jnp.take_along_axis(csum, ef[:, None], axis=1)[:, 0] - 1
    nb = (counts + MBLK - 1) // MBLK
    cumnb = jnp.cumsum(nb)
    poff = (cumnb - nb) * MBLK
    sortpos = poff[ef] + rank
    gidx = jnp.zeros((NPAD,), jnp.int32).at[sortpos].set(
        jnp.arange(2 * T, dtype=jnp.int32) // 2)
    gw_pad = jnp.zeros((NPAD,), jnp.float32).at[sortpos].set(w.reshape(-1))
    block_expert = jnp.minimum(
        jnp.searchsorted(cumnb, jnp.arange(NBLK), side="right"),
        NE - 1).astype(jnp.int32)
    pos12 = sortpos.reshape(T, 2).astype(jnp.int32)

    xg = _get_gather_sc()(x2d, gidx)
    contrib = _mlp(block_expert, xg, gw_pad.reshape(NBLK, MBLK, 1), egh, euh,
                   edh)
    out2d = _get_combine_sc()(contrib, pos12[:, 0], pos12[:, 1])
    return out2d.reshape(hidden_states.shape)
